# Initial kernel scaffold; baseline (speedup 1.0000x reference)
import jax
import jax.numpy as jnp
from jax.experimental import pallas as pl


def kernel(x, edge_index, edge_attr, batch, W1e, b1e, W2e, b2e, W1c1, b1c1, W2c1, b2c1, W1c2, b1c2, W2c2, b2c2, D1, bD1, D2, bD2, D3, bD3):
    raise NotImplementedError("write your pallas kernel here")


# speedup vs baseline: 1.5454x; 1.5454x over previous
import functools

import jax
import jax.numpy as jnp
from jax import lax
from jax.experimental import pallas as pl
from jax.experimental.pallas import tpu as pltpu
from jax.experimental.pallas import tpu_sc as plsc

N = 10000
E = 320000
NPAD = 10240
ROWS = 320
OWN_MUL = 6554
OWN_SHR = 21
NW = 32
GBLK = 128
NGB = E // GBLK
GB_PER_W = (NGB + NW - 1) // NW
SWIN = 2000
NSW = E // SWIN
MCH = 64
H = 128

_mesh = plsc.VectorSubcoreMesh(core_axis_name="c", subcore_axis_name="s")


def _wid():
    return lax.axis_index("s") * 2 + lax.axis_index("c")


def _gather_kernel(a_h, b_h, src_h, dst_h, g_h, idxs_v, idxd_v, buf_v, sem):
    wid = _wid()

    def step(i, carry):
        blk = wid + i * NW

        @pl.when(blk < NGB)
        def _():
            base = blk * GBLK
            pltpu.sync_copy(dst_h.at[pl.ds(base, GBLK)], idxd_v)
            pltpu.sync_copy(src_h.at[pl.ds(base, GBLK)], idxs_v)
            pltpu.async_copy(a_h.at[idxd_v], buf_v, sem).wait()
            pltpu.async_copy(b_h.at[idxs_v], buf_v, sem, add=True).wait()
            pltpu.sync_copy(buf_v, g_h.at[pl.ds(base, GBLK)])

        return carry

    lax.fori_loop(0, GB_PER_W, step, 0)


def _sc_gather(A, B, src, dst):
    f = pl.kernel(
        _gather_kernel,
        out_type=jax.ShapeDtypeStruct((E, H), jnp.float32),
        mesh=_mesh,
        scratch_types=[
            pltpu.VMEM((GBLK,), jnp.int32),
            pltpu.VMEM((GBLK,), jnp.int32),
            pltpu.VMEM((GBLK, H), jnp.float32),
            pltpu.SemaphoreType.DMA,
        ],
    )
    return f(A, B, src, dst)


def _scatter_kernel(m_h, dst_h, out_h, dstw_v, eid_v, lrow_v, mbuf_v, acc_v, sem):
    wid = _wid()
    iota16 = lax.iota(jnp.int32, 16)
    zeros16 = jnp.zeros((16,), jnp.float32)

    def zrow(i, carry):
        for k in range(8):
            acc_v[i, pl.ds(k * 16, 16)] = zeros16
        return carry

    lax.fori_loop(0, ROWS, zrow, 0)

    def window(w, carry):
        ebase = w * SWIN
        pltpu.sync_copy(dst_h.at[pl.ds(ebase, SWIN)], dstw_v)

        def scan_step(i, cnt):
            v = dstw_v[pl.ds(i * 16, 16)]
            own = lax.shift_right_arithmetic(v * OWN_MUL, OWN_SHR)
            msk = own == wid
            mi = msk.astype(jnp.int32)
            cs = plsc.cumsum(mi)
            pos = cnt + cs - mi
            eidv = ebase + i * 16 + iota16
            lrv = v - own * ROWS
            plsc.store_scatter(eid_v, [pos], eidv, mask=msk)
            plsc.store_scatter(lrow_v, [pos], lrv, mask=msk)
            return cnt + jnp.max(cs)

        cnt = lax.fori_loop(0, SWIN // 16, scan_step, jnp.int32(0))

        for k in range(MCH // 16):
            plsc.store_scatter(eid_v, [cnt + k * 16 + iota16], k * 16 + iota16)

        nch = lax.shift_right_arithmetic(cnt + (MCH - 1), 6)

        def chunk(ci, carry):
            cb = ci * MCH
            pltpu.async_copy(m_h.at[eid_v.at[pl.ds(cb, MCH)]], mbuf_v, sem).wait()
            nrows = jnp.minimum(cnt - cb, MCH)

            def rmw(j, c2):
                lr = lrow_v[pl.ds(cb + j, 16)][0]
                for k in range(8):
                    a = acc_v[lr, pl.ds(k * 16, 16)]
                    x = mbuf_v[j, pl.ds(k * 16, 16)]
                    acc_v[lr, pl.ds(k * 16, 16)] = jnp.maximum(a, x)
                return c2

            lax.fori_loop(0, nrows, rmw, 0)
            return carry

        lax.fori_loop(0, nch, chunk, 0)
        return carry

    lax.fori_loop(0, NSW, window, 0)
    pltpu.sync_copy(acc_v, out_h.at[pl.ds(wid * ROWS, ROWS)])


def _sc_scatter_max(M, dst):
    f = pl.kernel(
        _scatter_kernel,
        out_type=jax.ShapeDtypeStruct((NPAD, H), jnp.float32),
        mesh=_mesh,
        compiler_params=pltpu.CompilerParams(needs_layout_passes=False),
        scratch_types=[
            pltpu.VMEM((SWIN,), jnp.int32),
            pltpu.VMEM((SWIN + MCH + 16,), jnp.int32),
            pltpu.VMEM((SWIN + 16,), jnp.int32),
            pltpu.VMEM((MCH, H), jnp.float32),
            pltpu.VMEM((ROWS, H), jnp.float32),
            pltpu.SemaphoreType.DMA,
        ],
    )
    return f(M, dst)


def _node_body(h_ref, w1_ref, b1_ref, a_ref, b_ref, *, fin):
    h = h_ref[...]
    wtop = w1_ref[:fin, :]
    wbot = w1_ref[fin:, :]
    a_ref[...] = (
        jnp.dot(h, wtop - wbot, preferred_element_type=jnp.float32) + b1_ref[...]
    )
    b_ref[...] = jnp.dot(h, wbot, preferred_element_type=jnp.float32)


def _tc_node(h, W1, b1):
    npad, fin = h.shape
    blk = 512
    body = functools.partial(_node_body, fin=fin)
    return pl.pallas_call(
        body,
        grid=(npad // blk,),
        in_specs=[
            pl.BlockSpec((blk, fin), lambda i: (i, 0)),
            pl.BlockSpec((2 * fin, H), lambda i: (0, 0)),
            pl.BlockSpec((1, H), lambda i: (0, 0)),
        ],
        out_specs=[
            pl.BlockSpec((blk, H), lambda i: (i, 0)),
            pl.BlockSpec((blk, H), lambda i: (i, 0)),
        ],
        out_shape=[jax.ShapeDtypeStruct((npad, H), jnp.float32)] * 2,
    )(h, W1, b1.reshape(1, H))


def _edge_body(g_ref, w2_ref, b2_ref, m_ref):
    g = jnp.maximum(g_ref[...], 0.0)
    m_ref[...] = (
        jnp.dot(g, w2_ref[...], preferred_element_type=jnp.float32) + b2_ref[...]
    )


def _tc_edge_mm(G, W2, b2):
    blk = 2000
    return pl.pallas_call(
        _edge_body,
        grid=(E // blk,),
        in_specs=[
            pl.BlockSpec((blk, H), lambda i: (i, 0)),
            pl.BlockSpec((H, H), lambda i: (0, 0)),
            pl.BlockSpec((1, H), lambda i: (0, 0)),
        ],
        out_specs=pl.BlockSpec((blk, H), lambda i: (i, 0)),
        out_shape=jax.ShapeDtypeStruct((E, H), jnp.float32),
    )(G, W2, b2.reshape(1, H))


def _final_body(h_ref, d1_ref, bd1_ref, d2_ref, bd2_ref, d3_ref, bd3_ref, o_ref):
    z = jnp.max(h_ref[...], axis=0, keepdims=True)
    z = jnp.maximum(
        jnp.dot(z, d1_ref[...], preferred_element_type=jnp.float32) + bd1_ref[...], 0.0
    )
    z = jnp.maximum(
        jnp.dot(z, d2_ref[...], preferred_element_type=jnp.float32) + bd2_ref[...], 0.0
    )
    o_ref[...] = (
        jnp.dot(z, d3_ref[...], preferred_element_type=jnp.float32) + bd3_ref[...]
    )


def _tc_final(h3, D1, bD1, D2, bD2, D3, bD3):
    return pl.pallas_call(
        _final_body,
        out_shape=jax.ShapeDtypeStruct((1, 4), jnp.float32),
    )(h3, D1, bD1.reshape(1, -1), D2, bD2.reshape(1, -1), D3, bD3.reshape(1, -1))


def kernel(x, edge_index, edge_attr, batch,
           W1e, b1e, W2e, b2e,
           W1c1, b1c1, W2c1, b2c1,
           W1c2, b1c2, W2c2, b2c2,
           D1, bD1, D2, bD2, D3, bD3):
    src = edge_index[0]
    dst = edge_index[1]

    h = jnp.pad(x, ((0, NPAD - N), (0, 0)))
    for W1, b1, W2, b2 in (
        (W1e, b1e, W2e, b2e),
        (W1c1, b1c1, W2c1, b2c1),
        (W1c2, b1c2, W2c2, b2c2),
    ):
        A, B = _tc_node(h, W1, b1)
        G = _sc_gather(A, B, src, dst)
        M = _tc_edge_mm(G, W2, b2)
        h = _sc_scatter_max(M, dst)

    probs = _tc_final(h, D1, bD1, D2, bD2, D3, bD3)
    return (probs, edge_attr)


# speedup vs baseline: 2.2190x; 1.4358x over previous
import functools

import jax
import jax.numpy as jnp
from jax import lax
from jax.experimental import pallas as pl
from jax.experimental.pallas import tpu as pltpu
from jax.experimental.pallas import tpu_sc as plsc

N = 10000
E = 320000
NPAD = 10240
ROWS = 320
OWN_MUL = 6554
OWN_SHR = 21
NW = 32
GBLK = 128
NGB = E // GBLK
GB_PER_W = (NGB + NW - 1) // NW
MCH = 64
PWIN = 8000
NPW = E // PWIN
SLOT = PWIN + 128
CNTW = 64
H = 128

_mesh = plsc.VectorSubcoreMesh(core_axis_name="c", subcore_axis_name="s")


def _wid():
    return lax.axis_index("s") * 2 + lax.axis_index("c")


def _gather_kernel(a_h, b_h, src_h, dst_h, g_h, idxs_v, idxd_v, buf_v, sem):
    wid = _wid()

    def step(i, carry):
        blk = wid + i * NW

        @pl.when(blk < NGB)
        def _():
            base = blk * GBLK
            pltpu.sync_copy(dst_h.at[pl.ds(base, GBLK)], idxd_v)
            pltpu.sync_copy(src_h.at[pl.ds(base, GBLK)], idxs_v)
            pltpu.async_copy(a_h.at[idxd_v], buf_v, sem).wait()
            pltpu.async_copy(b_h.at[idxs_v], buf_v, sem, add=True).wait()
            pltpu.sync_copy(buf_v, g_h.at[pl.ds(base, GBLK)])

        return carry

    lax.fori_loop(0, GB_PER_W, step, 0)


def _sc_gather(A, B, src, dst):
    f = pl.kernel(
        _gather_kernel,
        out_type=jax.ShapeDtypeStruct((E, H), jnp.float32),
        mesh=_mesh,
        scratch_types=[
            pltpu.VMEM((GBLK,), jnp.int32),
            pltpu.VMEM((GBLK,), jnp.int32),
            pltpu.VMEM((GBLK, H), jnp.float32),
            pltpu.SemaphoreType.DMA,
        ],
    )
    return f(A, B, src, dst)


def _pre_kernel(dst_h, eid_h, cnt_h, dstw_v, eidw_v, cbuf_v):
    wid = _wid()
    iota16 = lax.iota(jnp.int32, 16)

    def window(win, carry):
        pltpu.sync_copy(dst_h.at[pl.ds(win * PWIN, PWIN)], dstw_v)

        def scan_step(i, cw):
            v = dstw_v[pl.ds(i * 16, 16)]
            own = lax.shift_right_arithmetic(v * OWN_MUL, OWN_SHR)
            msk = own == wid
            eidv = win * PWIN + i * 16 + iota16
            plsc.store_compressed(eidw_v.at[pl.ds(cw, 16)], eidv, mask=msk)
            pc = plsc.all_reduce_population_count(msk)
            return cw + (pc if pc.ndim == 0 else pc[0])

        cw = lax.fori_loop(0, PWIN // 16, scan_step, jnp.int32(0))

        for k in range(MCH // 16):
            eidw_v[pl.ds(cw + k * 16, 16)] = k * 16 + iota16

        plsc.store_scatter(
            cbuf_v,
            [jnp.zeros((16,), jnp.int32) + win],
            jnp.zeros((16,), jnp.int32) + cw,
            mask=iota16 == 0,
        )
        pltpu.sync_copy(eidw_v, eid_h.at[wid, win])
        return carry

    lax.fori_loop(0, NPW, window, 0)
    pltpu.sync_copy(cbuf_v, cnt_h.at[wid])


def _sc_preprocess(dst):
    f = pl.kernel(
        _pre_kernel,
        out_type=[
            jax.ShapeDtypeStruct((NW, NPW, SLOT), jnp.int32),
            jax.ShapeDtypeStruct((NW, CNTW), jnp.int32),
        ],
        mesh=_mesh,
        compiler_params=pltpu.CompilerParams(needs_layout_passes=False),
        scratch_types=[
            pltpu.VMEM((PWIN,), jnp.int32),
            pltpu.VMEM((SLOT,), jnp.int32),
            pltpu.VMEM((CNTW,), jnp.int32),
        ],
    )
    return f(dst)


def _scatter_kernel(m_h, dst_h, eid_h, cnt_h, out_h,
                    eidw_v, cbuf_v, mb0, mb1, dv0, dv1, acc_v, sem0, sem1):
    wid = _wid()
    zeros16 = jnp.zeros((16,), jnp.float32)
    wbase = wid * ROWS

    def zrow(i, carry):
        for k in range(8):
            acc_v[i, pl.ds(k * 16, 16)] = zeros16
        return carry

    lax.fori_loop(0, ROWS, zrow, 0)
    pltpu.sync_copy(cnt_h.at[wid], cbuf_v)

    def start(ci, mb, dv, sem):
        idx = eidw_v.at[pl.ds(ci * MCH, MCH)]
        pltpu.async_copy(m_h.at[idx], mb, sem)
        pltpu.async_copy(dst_h.at[idx], dv.at[pl.ds(0, MCH)], sem)

    def wait(mb, dv, sem):
        idx = eidw_v.at[pl.ds(0, MCH)]
        pltpu.make_async_copy(m_h.at[idx], mb, sem).wait()
        pltpu.make_async_copy(dst_h.at[idx], dv.at[pl.ds(0, MCH)], sem).wait()

    def window(win, carry):
        pltpu.sync_copy(eid_h.at[wid, win], eidw_v)
        cw = cbuf_v[pl.ds(win, 16)][0]
        nch = lax.shift_right_arithmetic(cw + (MCH - 1), 6)

        @pl.when(nch > 0)
        def _():
            start(0, mb0, dv0, sem0)

        npair = lax.shift_right_arithmetic(nch + 1, 1)

        def pair(pi, carry2):
            for p in (0, 1):
                mb, dv, sem = (mb0, dv0, sem0) if p == 0 else (mb1, dv1, sem1)
                mbn, dvn, semn = (mb1, dv1, sem1) if p == 0 else (mb0, dv0, sem0)
                ci = pi * 2 + p

                @pl.when(ci < nch)
                def _(ci=ci, mb=mb, dv=dv, sem=sem, mbn=mbn, dvn=dvn, semn=semn):
                    @pl.when(ci + 1 < nch)
                    def _():
                        start(ci + 1, mbn, dvn, semn)

                    wait(mb, dv, sem)
                    cb = ci * MCH
                    nrows = jnp.minimum(cw - cb, MCH)

                    def rmw(j, c3):
                        lr = dv[pl.ds(j, 16)][0] - wbase
                        for k in range(8):
                            a = acc_v[lr, pl.ds(k * 16, 16)]
                            x = mb[j, pl.ds(k * 16, 16)]
                            acc_v[lr, pl.ds(k * 16, 16)] = jnp.maximum(a, x)
                        return c3

                    lax.fori_loop(0, nrows, rmw, 0)

            return carry2

        lax.fori_loop(0, npair, pair, 0)
        return carry

    lax.fori_loop(0, NPW, window, 0)
    pltpu.sync_copy(acc_v, out_h.at[pl.ds(wbase, ROWS)])


def _sc_scatter_max(M, dst, eid_slots, counts):
    f = pl.kernel(
        _scatter_kernel,
        out_type=jax.ShapeDtypeStruct((NPAD, H), jnp.float32),
        mesh=_mesh,
        compiler_params=pltpu.CompilerParams(needs_layout_passes=False),
        scratch_types=[
            pltpu.VMEM((SLOT,), jnp.int32),
            pltpu.VMEM((CNTW,), jnp.int32),
            pltpu.VMEM((MCH, H), jnp.float32),
            pltpu.VMEM((MCH, H), jnp.float32),
            pltpu.VMEM((MCH + 16,), jnp.int32),
            pltpu.VMEM((MCH + 16,), jnp.int32),
            pltpu.VMEM((ROWS, H), jnp.float32),
            pltpu.SemaphoreType.DMA,
            pltpu.SemaphoreType.DMA,
        ],
    )
    return f(M, dst, eid_slots, counts)


def _node_body(h_ref, w1_ref, b1_ref, a_ref, b_ref, *, fin):
    h = h_ref[...]
    wtop = w1_ref[:fin, :]
    wbot = w1_ref[fin:, :]
    a_ref[...] = (
        jnp.dot(h, wtop - wbot, preferred_element_type=jnp.float32) + b1_ref[...]
    )
    b_ref[...] = jnp.dot(h, wbot, preferred_element_type=jnp.float32)


def _tc_node(h, W1, b1):
    npad, fin = h.shape
    blk = 512
    body = functools.partial(_node_body, fin=fin)
    return pl.pallas_call(
        body,
        grid=(npad // blk,),
        in_specs=[
            pl.BlockSpec((blk, fin), lambda i: (i, 0)),
            pl.BlockSpec((2 * fin, H), lambda i: (0, 0)),
            pl.BlockSpec((1, H), lambda i: (0, 0)),
        ],
        out_specs=[
            pl.BlockSpec((blk, H), lambda i: (i, 0)),
            pl.BlockSpec((blk, H), lambda i: (i, 0)),
        ],
        out_shape=[jax.ShapeDtypeStruct((npad, H), jnp.float32)] * 2,
    )(h, W1, b1.reshape(1, H))


def _edge_body(g_ref, w2_ref, b2_ref, m_ref):
    g = jnp.maximum(g_ref[...], 0.0)
    m_ref[...] = (
        jnp.dot(g, w2_ref[...], preferred_element_type=jnp.float32) + b2_ref[...]
    )


def _tc_edge_mm(G, W2, b2):
    blk = 2000
    return pl.pallas_call(
        _edge_body,
        grid=(E // blk,),
        in_specs=[
            pl.BlockSpec((blk, H), lambda i: (i, 0)),
            pl.BlockSpec((H, H), lambda i: (0, 0)),
            pl.BlockSpec((1, H), lambda i: (0, 0)),
        ],
        out_specs=pl.BlockSpec((blk, H), lambda i: (i, 0)),
        out_shape=jax.ShapeDtypeStruct((E, H), jnp.float32),
    )(G, W2, b2.reshape(1, H))


def _final_body(h_ref, d1_ref, bd1_ref, d2_ref, bd2_ref, d3_ref, bd3_ref, o_ref):
    z = jnp.max(h_ref[...], axis=0, keepdims=True)
    z = jnp.maximum(
        jnp.dot(z, d1_ref[...], preferred_element_type=jnp.float32) + bd1_ref[...], 0.0
    )
    z = jnp.maximum(
        jnp.dot(z, d2_ref[...], preferred_element_type=jnp.float32) + bd2_ref[...], 0.0
    )
    o_ref[...] = (
        jnp.dot(z, d3_ref[...], preferred_element_type=jnp.float32) + bd3_ref[...]
    )


def _tc_final(h3, D1, bD1, D2, bD2, D3, bD3):
    return pl.pallas_call(
        _final_body,
        out_shape=jax.ShapeDtypeStruct((1, 4), jnp.float32),
    )(h3, D1, bD1.reshape(1, -1), D2, bD2.reshape(1, -1), D3, bD3.reshape(1, -1))


def kernel(x, edge_index, edge_attr, batch,
           W1e, b1e, W2e, b2e,
           W1c1, b1c1, W2c1, b2c1,
           W1c2, b1c2, W2c2, b2c2,
           D1, bD1, D2, bD2, D3, bD3):
    src = edge_index[0]
    dst = edge_index[1]

    eid_slots, counts = _sc_preprocess(dst)

    h = jnp.pad(x, ((0, NPAD - N), (0, 0)))
    for W1, b1, W2, b2 in (
        (W1e, b1e, W2e, b2e),
        (W1c1, b1c1, W2c1, b2c1),
        (W1c2, b1c2, W2c2, b2c2),
    ):
        A, B = _tc_node(h, W1, b1)
        G = _sc_gather(A, B, src, dst)
        M = _tc_edge_mm(G, W2, b2)
        h = _sc_scatter_max(M, dst, eid_slots, counts)

    probs = _tc_final(h, D1, bD1, D2, bD2, D3, bD3)
    return (probs, edge_attr)
